# baseline (speedup 1.0000x reference)
import functools

import jax
import jax.numpy as jnp
from jax import lax
from jax.experimental import pallas as pl
from jax.experimental.pallas import tpu as pltpu, tpu_sc as plsc

DIM = 128
N_CODES = 8192
N_TOK = 64 * 576

TM = 512
TN = 512
N_CHUNKS = N_CODES // TN
GRID = N_TOK // TM

def _tc_body(z_ref, et_ref, idx_ref, loss_ref):
    zb = z_ref[...]
    zf2 = jnp.sum(zb * zb, axis=1, keepdims=True)
    n = jnp.sqrt(zf2)
    zn = zb / jnp.maximum(n, 1e-12)
    zn2 = jnp.sum(zn * zn, axis=1, keepdims=True)

    def step(c, carry):
        minv, mini = carry
        eb = et_ref[c]
        e2 = jnp.sum(eb * eb, axis=0, keepdims=True)
        mm = lax.dot_general(zn, eb, (((1,), (0,)), ((), ())),
                             preferred_element_type=jnp.float32)
        d = (zn2 + e2) - 2.0 * mm
        cmin = jnp.min(d, axis=1, keepdims=True)
        iota = lax.broadcasted_iota(jnp.int32, (TM, TN), 1)
        cidx = jnp.min(jnp.where(d == cmin, iota, 2**30),
                       axis=1, keepdims=True) + c * TN
        take = cmin < minv
        return jnp.where(take, cmin, minv), jnp.where(take, cidx, mini)

    minv0 = jnp.full((TM, 1), jnp.inf, jnp.float32)
    mini0 = jnp.zeros((TM, 1), jnp.int32)
    minv, mini = lax.fori_loop(0, N_CHUNKS, step, (minv0, mini0))

    idx_ref[...] = mini
    blocksum = jnp.sum(minv, axis=0, keepdims=True)

    @pl.when(pl.program_id(0) == 0)
    def _init():
        loss_ref[...] = blocksum

    @pl.when(pl.program_id(0) != 0)
    def _acc():
        loss_ref[...] += blocksum


_tc_call = pl.pallas_call(
    _tc_body,
    grid=(GRID,),
    in_specs=[
        pl.BlockSpec((TM, DIM), lambda i: (i, 0)),
        pl.BlockSpec((N_CHUNKS, DIM, TN), lambda i: (0, 0, 0)),
    ],
    out_specs=[
        pl.BlockSpec((TM, 1), lambda i: (i, 0)),
        pl.BlockSpec((1, 1), lambda i: (0, 0)),
    ],
    out_shape=[
        jax.ShapeDtypeStruct((N_TOK, 1), jnp.int32),
        jax.ShapeDtypeStruct((1, 1), jnp.float32),
    ],
)


_NUM_SC = 2
_NUM_SUBCORES = 16
_NW = _NUM_SC * _NUM_SUBCORES
_B_PER_W = N_TOK // _NW
_GCHUNK = 128
_N_GCHUNK = _B_PER_W // _GCHUNK


@functools.cache
def _sc_gather_kernel():
    @functools.partial(
        pl.kernel,
        out_type=jax.ShapeDtypeStruct((N_TOK, DIM), jnp.float32),
        mesh=plsc.VectorSubcoreMesh(core_axis_name="c", subcore_axis_name="s"),
        scratch_types=[
            pltpu.VMEM((_GCHUNK,), jnp.int32),
            pltpu.VMEM((_GCHUNK, DIM), jnp.float32),
            pltpu.SemaphoreType.DMA,
        ],
    )
    def _sc_gather(table_hbm, idx_hbm, out_hbm, idx_v, rows_v, sem):
        wid = lax.axis_index("s") * _NUM_SC + lax.axis_index("c")
        base = wid * _B_PER_W

        def chunk(c, carry):
            off = base + c * _GCHUNK
            pltpu.sync_copy(idx_hbm.at[pl.ds(off, _GCHUNK)], idx_v)
            pltpu.async_copy(table_hbm.at[idx_v], rows_v, sem).wait()
            pltpu.sync_copy(rows_v, out_hbm.at[pl.ds(off, _GCHUNK)])
            return carry

        lax.fori_loop(0, _N_GCHUNK, chunk, 0)

    return _sc_gather


def kernel(z, embedding):
    zf = z.reshape(N_TOK, DIM)
    et = embedding.reshape(N_CHUNKS, TN, DIM).transpose(0, 2, 1)
    idx2, losssum = _tc_call(zf, et)
    idx = idx2.reshape(N_TOK)
    z_q = _sc_gather_kernel()(embedding, idx).reshape(z.shape)
    loss = 1.0 * (losssum[0, 0] / (N_TOK * DIM))
    return (z_q, loss, idx)


# speedup vs baseline: 1.2604x; 1.2604x over previous
import functools

import jax
import jax.numpy as jnp
from jax import lax
from jax.experimental import pallas as pl
from jax.experimental.pallas import tpu as pltpu, tpu_sc as plsc

DIM = 128
N_CODES = 8192
N_TOK = 64 * 576

TM = 512
TN = 512
N_CHUNKS = N_CODES // TN
GRID = N_TOK // TM

N_TILES = TN // 8


def _tc_body(z_ref, e_ref, idx_ref, loss_ref, e2_ref):
    @pl.when(pl.program_id(0) == 0)
    def _precompute_e2():
        def e2c(c, carry):
            eb = e_ref[c]
            e2_ref[c] = jnp.sum(eb * eb, axis=1, keepdims=True)
            return carry
        lax.fori_loop(0, N_CHUNKS, e2c, 0)

    zb = z_ref[...]
    zf2 = jnp.sum(zb * zb, axis=1, keepdims=True)
    n = jnp.sqrt(zf2)
    zn = zb / jnp.maximum(n, 1e-12)
    zn2 = jnp.sum(zn * zn, axis=1, keepdims=True)
    znt = jnp.transpose(zn * (-2.0))

    iota_t = lax.broadcasted_iota(jnp.int32, (N_TILES, 8, TM), 0)

    def step(c, carry):
        av, ai = carry
        eb = e_ref[c]
        mm = lax.dot_general(eb, znt, (((1,), (0,)), ((), ())),
                             preferred_element_type=jnp.float32)
        d3 = (mm + e2_ref[c]).reshape(N_TILES, 8, TM)
        cmin = jnp.min(d3, axis=0)
        tloc = jnp.min(jnp.where(d3 == cmin[None], iota_t, 2**30), axis=0)
        take = cmin < av
        return (jnp.where(take, cmin, av),
                jnp.where(take, c * N_TILES + tloc, ai))

    av0 = jnp.full((8, TM), jnp.inf, jnp.float32)
    ai0 = jnp.zeros((8, TM), jnp.int32)
    av, ai = lax.fori_loop(0, N_CHUNKS, step, (av0, ai0))

    gidx = ai * 8 + lax.broadcasted_iota(jnp.int32, (8, TM), 0)
    vmin = jnp.min(av, axis=0, keepdims=True)
    imin = jnp.min(jnp.where(av == vmin, gidx, 2**30), axis=0, keepdims=True)
    idx_ref[...] = imin.reshape(1, 1, TM)

    blocksum = (jnp.sum(vmin, axis=1, keepdims=True)
                + jnp.sum(zn2, axis=0, keepdims=True))

    @pl.when(pl.program_id(0) == 0)
    def _init():
        loss_ref[...] = blocksum

    @pl.when(pl.program_id(0) != 0)
    def _acc():
        loss_ref[...] += blocksum


_tc_call = pl.pallas_call(
    _tc_body,
    grid=(GRID,),
    in_specs=[
        pl.BlockSpec((TM, DIM), lambda i: (i, 0)),
        pl.BlockSpec((N_CHUNKS, TN, DIM), lambda i: (0, 0, 0)),
    ],
    out_specs=[
        pl.BlockSpec((1, 1, TM), lambda i: (i, 0, 0)),
        pl.BlockSpec((1, 1), lambda i: (0, 0)),
    ],
    out_shape=[
        jax.ShapeDtypeStruct((GRID, 1, TM), jnp.int32),
        jax.ShapeDtypeStruct((1, 1), jnp.float32),
    ],
    scratch_shapes=[pltpu.VMEM((N_CHUNKS, TN, 1), jnp.float32)],
)


_NUM_SC = 2
_NUM_SUBCORES = 16
_NW = _NUM_SC * _NUM_SUBCORES
_B_PER_W = N_TOK // _NW
_GCHUNK = 128
_N_GCHUNK = _B_PER_W // _GCHUNK


@functools.cache
def _sc_gather_kernel():
    @functools.partial(
        pl.kernel,
        out_type=jax.ShapeDtypeStruct((N_TOK, DIM), jnp.float32),
        mesh=plsc.VectorSubcoreMesh(core_axis_name="c", subcore_axis_name="s"),
        scratch_types=[
            pltpu.VMEM((_GCHUNK,), jnp.int32),
            pltpu.VMEM((_GCHUNK, DIM), jnp.float32),
            pltpu.SemaphoreType.DMA,
        ],
    )
    def _sc_gather(table_hbm, idx_hbm, out_hbm, idx_v, rows_v, sem):
        wid = lax.axis_index("s") * _NUM_SC + lax.axis_index("c")
        base = wid * _B_PER_W

        def chunk(c, carry):
            off = base + c * _GCHUNK
            pltpu.sync_copy(idx_hbm.at[pl.ds(off, _GCHUNK)], idx_v)
            pltpu.async_copy(table_hbm.at[idx_v], rows_v, sem).wait()
            pltpu.sync_copy(rows_v, out_hbm.at[pl.ds(off, _GCHUNK)])
            return carry

        lax.fori_loop(0, _N_GCHUNK, chunk, 0)

    return _sc_gather


def kernel(z, embedding):
    zf = z.reshape(N_TOK, DIM)
    e3 = embedding.reshape(N_CHUNKS, TN, DIM)
    idx2, losssum = _tc_call(zf, e3)
    idx = idx2.reshape(N_TOK)
    z_q = _sc_gather_kernel()(embedding, idx).reshape(z.shape)
    loss = 1.0 * (losssum[0, 0] / (N_TOK * DIM))
    return (z_q, loss, idx)


# speedup vs baseline: 1.7920x; 1.4217x over previous
import functools

import jax
import jax.numpy as jnp
from jax import lax
from jax.experimental import pallas as pl
from jax.experimental.pallas import tpu as pltpu, tpu_sc as plsc

DIM = 128
N_CODES = 8192
N_TOK = 64 * 576

TM = 512
TN = 512
N_CHUNKS = N_CODES // TN
GRID = N_TOK // TM

N_TILES = TN // 8


def _tc_body(z_ref, e_ref, idx_ref, loss_ref, e2_ref):
    @pl.when(pl.program_id(0) == 0)
    def _precompute_e2():
        def e2c(c, carry):
            eb = e_ref[c]
            e2_ref[c] = jnp.sum(eb * eb, axis=1, keepdims=True)
            return carry
        lax.fori_loop(0, N_CHUNKS, e2c, 0)

    zb = z_ref[...]
    zf2 = jnp.sum(zb * zb, axis=1, keepdims=True)
    n = jnp.sqrt(zf2)
    zn = zb / jnp.maximum(n, 1e-12)
    zn2 = jnp.sum(zn * zn, axis=1, keepdims=True)
    znt = jnp.transpose(zn * (-2.0))

    iota_t = lax.broadcasted_iota(jnp.int32, (N_TILES, 8, TM), 0).astype(jnp.float32)

    def step(c, carry):
        av, ai = carry
        eb = e_ref[c]
        mm = lax.dot_general(eb, znt, (((1,), (0,)), ((), ())),
                             preferred_element_type=jnp.float32)
        d3 = (mm + e2_ref[c]).reshape(N_TILES, 8, TM)
        cmin = jnp.min(d3, axis=0)
        tloc = jnp.min(jnp.where(d3 == cmin[None], iota_t, 2.0**30), axis=0)
        take = cmin < av
        return (jnp.where(take, cmin, av),
                jnp.where(take, c * jnp.float32(N_TILES) + tloc, ai))

    av0 = jnp.full((8, TM), jnp.inf, jnp.float32)
    ai0 = jnp.zeros((8, TM), jnp.float32)
    av, ai = lax.fori_loop(0, N_CHUNKS, step, (av0, ai0), unroll=2)

    gidx = (ai.astype(jnp.int32) * 8
            + lax.broadcasted_iota(jnp.int32, (8, TM), 0))
    vmin = jnp.min(av, axis=0, keepdims=True)
    imin = jnp.min(jnp.where(av == vmin, gidx, 2**30), axis=0, keepdims=True)
    idx_ref[...] = imin.reshape(1, 1, TM)

    blocksum = (jnp.sum(vmin, axis=1, keepdims=True)
                + jnp.sum(zn2, axis=0, keepdims=True))

    @pl.when(pl.program_id(0) == 0)
    def _init():
        loss_ref[...] = blocksum

    @pl.when(pl.program_id(0) != 0)
    def _acc():
        loss_ref[...] += blocksum


_tc_call = pl.pallas_call(
    _tc_body,
    grid=(GRID,),
    in_specs=[
        pl.BlockSpec((TM, DIM), lambda i: (i, 0)),
        pl.BlockSpec((N_CHUNKS, TN, DIM), lambda i: (0, 0, 0)),
    ],
    out_specs=[
        pl.BlockSpec((1, 1, TM), lambda i: (i, 0, 0)),
        pl.BlockSpec((1, 1), lambda i: (0, 0)),
    ],
    out_shape=[
        jax.ShapeDtypeStruct((GRID, 1, TM), jnp.int32),
        jax.ShapeDtypeStruct((1, 1), jnp.float32),
    ],
    scratch_shapes=[pltpu.VMEM((N_CHUNKS, TN, 1), jnp.float32)],
)


_NUM_SC = 2
_NUM_SUBCORES = 16
_NW = _NUM_SC * _NUM_SUBCORES
_B_PER_W = N_TOK // _NW
_GCHUNK = 128
_N_GCHUNK = _B_PER_W // _GCHUNK


@functools.cache
def _sc_gather_kernel():
    @functools.partial(
        pl.kernel,
        out_type=jax.ShapeDtypeStruct((N_TOK, DIM), jnp.float32),
        mesh=plsc.VectorSubcoreMesh(core_axis_name="c", subcore_axis_name="s"),
        scratch_types=[
            pltpu.VMEM((_GCHUNK,), jnp.int32),
            pltpu.VMEM((_GCHUNK, DIM), jnp.float32),
            pltpu.SemaphoreType.DMA,
        ],
    )
    def _sc_gather(table_hbm, idx_hbm, out_hbm, idx_v, rows_v, sem):
        wid = lax.axis_index("s") * _NUM_SC + lax.axis_index("c")
        base = wid * _B_PER_W

        def chunk(c, carry):
            off = base + c * _GCHUNK
            pltpu.sync_copy(idx_hbm.at[pl.ds(off, _GCHUNK)], idx_v)
            pltpu.async_copy(table_hbm.at[idx_v], rows_v, sem).wait()
            pltpu.sync_copy(rows_v, out_hbm.at[pl.ds(off, _GCHUNK)])
            return carry

        lax.fori_loop(0, _N_GCHUNK, chunk, 0)

    return _sc_gather


def kernel(z, embedding):
    zf = z.reshape(N_TOK, DIM)
    e3 = embedding.reshape(N_CHUNKS, TN, DIM)
    idx2, losssum = _tc_call(zf, e3)
    idx = idx2.reshape(N_TOK)
    z_q = _sc_gather_kernel()(embedding, idx).reshape(z.shape)
    loss = 1.0 * (losssum[0, 0] / (N_TOK * DIM))
    return (z_q, loss, idx)
